# baseline (speedup 1.0000x reference)
import functools

import jax
import jax.numpy as jnp
from jax import lax
from jax.experimental import pallas as pl
from jax.experimental.pallas import tpu as pltpu
from jax.experimental.pallas import tpu_sc as plsc

NUM_CORES = 2
NUM_SUBCORES = 16
LANES = 16
NW = NUM_CORES * NUM_SUBCORES
BATCH = 16384
EDIM = 32
NROW = 1000000
WIDE = 128
PACK = WIDE // EDIM
WROWS = NROW * EDIM // WIDE
N4 = 250112
NBLK = N4 // 128
INBLK = (NROW + 127) // 128 - 1
B_PER_W = BATCH // NW
CHUNK = 128
NCHUNK = B_PER_W // CHUNK
GROUPS = CHUNK // LANES


def _tc_relayout(t_t):
    def body(x0_ref, x1_ref, x2_ref, x3_ref, o_ref):
        for u, x_ref in enumerate((x0_ref, x1_ref, x2_ref, x3_ref)):
            o_ref[:, u * EDIM:(u + 1) * EDIM] = x_ref[...].T

    def in_spec(u):
        return pl.BlockSpec(
            (EDIM, 128),
            lambda g, u=u: (0, jnp.minimum(u * NBLK + g, INBLK)))

    return pl.pallas_call(
        body,
        grid=(NBLK,),
        in_specs=[in_spec(u) for u in range(4)],
        out_specs=pl.BlockSpec((128, WIDE), lambda g: (g, 0)),
        out_shape=jax.ShapeDtypeStruct((N4, WIDE), jnp.float32),
    )(t_t, t_t, t_t, t_t)


def _sc_pred_diff(u, i, j, tw_u, tw_i):
    mesh = plsc.VectorSubcoreMesh(core_axis_name="c", subcore_axis_name="s")

    @functools.partial(
        pl.kernel,
        out_type=jax.ShapeDtypeStruct((BATCH // CHUNK, CHUNK), jnp.float32),
        mesh=mesh,
        compiler_params=pltpu.CompilerParams(
            needs_layout_passes=False, use_tc_tiling_on_sc=False),
        scratch_types=[
            pltpu.VMEM((NCHUNK, CHUNK), jnp.int32),
            pltpu.VMEM((NCHUNK, CHUNK), jnp.int32),
            pltpu.VMEM((NCHUNK, CHUNK), jnp.int32),
            pltpu.VMEM((NCHUNK, CHUNK), jnp.int32),
            pltpu.VMEM((NCHUNK, CHUNK), jnp.int32),
            pltpu.VMEM((NCHUNK, CHUNK), jnp.int32),
            pltpu.VMEM((CHUNK, WIDE), jnp.float32),
            pltpu.VMEM((CHUNK, WIDE), jnp.float32),
            pltpu.VMEM((CHUNK, WIDE), jnp.float32),
            pltpu.VMEM((NCHUNK, CHUNK), jnp.float32),
            pltpu.SemaphoreType.DMA,
        ],
    )
    def run(u_hbm, i_hbm, j_hbm, ut_hbm, it_hbm, out_hbm,
            u_idx, i_idx, j_idx, uq, iq, jq, ue_v, ie_v, je_v, pred_v, sem):
        wid = lax.axis_index("s") * NUM_CORES + lax.axis_index("c")
        base = wid * B_PER_W
        for c in range(NCHUNK):
            src = pl.ds(base + c * CHUNK, CHUNK)
            pltpu.sync_copy(u_hbm.at[src], u_idx.at[c])
            pltpu.sync_copy(i_hbm.at[src], i_idx.at[c])
            pltpu.sync_copy(j_hbm.at[src], j_idx.at[c])

        def qbody(v, carry):
            s = pl.ds(pl.multiple_of(v * LANES, LANES), LANES)
            for src_ref, dst_ref in ((u_idx, uq), (i_idx, iq), (j_idx, jq)):
                for c in range(NCHUNK):
                    r = src_ref[c, s]
                    slot = r // N4
                    dst_ref[c, s] = r - slot * N4
                    src_ref[c, s] = slot * EDIM
            return carry

        lax.fori_loop(0, CHUNK // LANES, qbody, 0)

        lane = lax.iota(jnp.int32, LANES)
        for c in range(NCHUNK):
            cp_u = pltpu.async_copy(ut_hbm.at[uq.at[c]], ue_v, sem)
            cp_i = pltpu.async_copy(it_hbm.at[iq.at[c]], ie_v, sem)
            cp_j = pltpu.async_copy(it_hbm.at[jq.at[c]], je_v, sem)
            cp_u.wait()
            cp_i.wait()
            cp_j.wait()

            def body(g, carry, c=c):
                s = pl.ds(pl.multiple_of(g * LANES, LANES), LANES)
                lrows = pl.multiple_of(g * LANES, LANES) + lane
                cb_u = u_idx[c, s]
                cb_i = i_idx[c, s]
                cb_j = j_idx[c, s]
                acc = jnp.zeros((LANES,), jnp.float32)
                for d in range(EDIM):
                    uev = plsc.load_gather(ue_v, [lrows, cb_u + d])
                    iev = plsc.load_gather(ie_v, [lrows, cb_i + d])
                    jev = plsc.load_gather(je_v, [lrows, cb_j + d])
                    acc = acc + uev * (iev - jev)
                pred_v[c, s] = acc
                return carry

            lax.fori_loop(0, GROUPS, body, 0)
        pltpu.sync_copy(pred_v, out_hbm.at[pl.ds(wid * NCHUNK, NCHUNK)])

    return run(u, i, j, tw_u, tw_i)


def _tc_loss(d2):
    def body(x_ref, o_ref):
        x = x_ref[...]
        ls = jnp.minimum(x, 0.0) - jnp.log(1.0 + jnp.exp(-jnp.abs(x)))
        o_ref[0, 0] = -jnp.sum(ls)

    out = pl.pallas_call(
        body,
        out_shape=jax.ShapeDtypeStruct((1, 1), jnp.float32),
        out_specs=pl.BlockSpec(memory_space=pltpu.SMEM),
    )(d2)
    return out[0, 0]


def kernel(u, i, j, user_embed, item_embed):
    tw_u = _tc_relayout(user_embed.T)
    tw_i = _tc_relayout(item_embed.T)
    d2 = _sc_pred_diff(u.astype(jnp.int32), i.astype(jnp.int32),
                       j.astype(jnp.int32), tw_u, tw_i)
    return _tc_loss(d2)


# speedup vs baseline: 3.6341x; 3.6341x over previous
import functools

import jax
import jax.numpy as jnp
from jax import lax
from jax.experimental import pallas as pl
from jax.experimental.pallas import tpu as pltpu
from jax.experimental.pallas import tpu_sc as plsc

NUM_CORES = 2
NUM_SUBCORES = 16
LANES = 16
NW = NUM_CORES * NUM_SUBCORES
BATCH = 16384
EDIM = 32
NROW = 1000000
WIDE = 128
PACK = WIDE // EDIM
WROWS = NROW * EDIM // WIDE
N4 = 250880
QB = 1024
NBLK = N4 // QB
INBLK = (NROW + QB - 1) // QB - 1
B_PER_W = BATCH // NW
CHUNK = 128
NCHUNK = B_PER_W // CHUNK
GROUPS = CHUNK // LANES


def _tc_relayout(t_t):
    def body(x0_ref, x1_ref, x2_ref, x3_ref, o_ref):
        for u, x_ref in enumerate((x0_ref, x1_ref, x2_ref, x3_ref)):
            o_ref[:, u * EDIM:(u + 1) * EDIM] = x_ref[...].T

    def in_spec(u):
        return pl.BlockSpec(
            (EDIM, QB),
            lambda g, u=u: (0, jnp.minimum(u * NBLK + g, INBLK)))

    return pl.pallas_call(
        body,
        grid=(NBLK,),
        in_specs=[in_spec(u) for u in range(4)],
        out_specs=pl.BlockSpec((QB, WIDE), lambda g: (g, 0)),
        out_shape=jax.ShapeDtypeStruct((N4, WIDE), jnp.float32),
    )(t_t, t_t, t_t, t_t)


def _sc_pred_diff(u, i, j, tw_u, tw_i):
    mesh = plsc.VectorSubcoreMesh(core_axis_name="c", subcore_axis_name="s")

    @functools.partial(
        pl.kernel,
        out_type=jax.ShapeDtypeStruct((BATCH // CHUNK, CHUNK), jnp.float32),
        mesh=mesh,
        compiler_params=pltpu.CompilerParams(
            needs_layout_passes=False, use_tc_tiling_on_sc=False),
        scratch_types=[
            pltpu.VMEM((NCHUNK, CHUNK), jnp.int32),
            pltpu.VMEM((NCHUNK, CHUNK), jnp.int32),
            pltpu.VMEM((NCHUNK, CHUNK), jnp.int32),
            pltpu.VMEM((NCHUNK, CHUNK), jnp.int32),
            pltpu.VMEM((NCHUNK, CHUNK), jnp.int32),
            pltpu.VMEM((NCHUNK, CHUNK), jnp.int32),
            pltpu.VMEM((CHUNK, WIDE), jnp.float32),
            pltpu.VMEM((CHUNK, WIDE), jnp.float32),
            pltpu.VMEM((CHUNK, WIDE), jnp.float32),
            pltpu.VMEM((NCHUNK, CHUNK), jnp.float32),
            pltpu.SemaphoreType.DMA,
        ],
    )
    def run(u_hbm, i_hbm, j_hbm, ut_hbm, it_hbm, out_hbm,
            u_idx, i_idx, j_idx, uq, iq, jq, ue_v, ie_v, je_v, pred_v, sem):
        wid = lax.axis_index("s") * NUM_CORES + lax.axis_index("c")
        base = wid * B_PER_W
        for c in range(NCHUNK):
            src = pl.ds(base + c * CHUNK, CHUNK)
            pltpu.sync_copy(u_hbm.at[src], u_idx.at[c])
            pltpu.sync_copy(i_hbm.at[src], i_idx.at[c])
            pltpu.sync_copy(j_hbm.at[src], j_idx.at[c])

        def qbody(v, carry):
            s = pl.ds(pl.multiple_of(v * LANES, LANES), LANES)
            for src_ref, dst_ref in ((u_idx, uq), (i_idx, iq), (j_idx, jq)):
                for c in range(NCHUNK):
                    r = src_ref[c, s]
                    slot = r // N4
                    dst_ref[c, s] = r - slot * N4
                    src_ref[c, s] = slot * EDIM
            return carry

        lax.fori_loop(0, CHUNK // LANES, qbody, 0)

        lane = lax.iota(jnp.int32, LANES)
        for c in range(NCHUNK):
            cp_u = pltpu.async_copy(ut_hbm.at[uq.at[c]], ue_v, sem)
            cp_i = pltpu.async_copy(it_hbm.at[iq.at[c]], ie_v, sem)
            cp_j = pltpu.async_copy(it_hbm.at[jq.at[c]], je_v, sem)
            cp_u.wait()
            cp_i.wait()
            cp_j.wait()

            def body(g, carry, c=c):
                s = pl.ds(pl.multiple_of(g * LANES, LANES), LANES)
                lrows = pl.multiple_of(g * LANES, LANES) + lane
                cb_u = u_idx[c, s]
                cb_i = i_idx[c, s]
                cb_j = j_idx[c, s]
                acc = jnp.zeros((LANES,), jnp.float32)
                for d in range(EDIM):
                    uev = plsc.load_gather(ue_v, [lrows, cb_u + d])
                    iev = plsc.load_gather(ie_v, [lrows, cb_i + d])
                    jev = plsc.load_gather(je_v, [lrows, cb_j + d])
                    acc = acc + uev * (iev - jev)
                pred_v[c, s] = acc
                return carry

            lax.fori_loop(0, GROUPS, body, 0)
        pltpu.sync_copy(pred_v, out_hbm.at[pl.ds(wid * NCHUNK, NCHUNK)])

    return run(u, i, j, tw_u, tw_i)


def _tc_loss(d2):
    def body(x_ref, o_ref):
        x = x_ref[...]
        ls = jnp.minimum(x, 0.0) - jnp.log(1.0 + jnp.exp(-jnp.abs(x)))
        o_ref[0, 0] = -jnp.sum(ls)

    out = pl.pallas_call(
        body,
        out_shape=jax.ShapeDtypeStruct((1, 1), jnp.float32),
        out_specs=pl.BlockSpec(memory_space=pltpu.SMEM),
    )(d2)
    return out[0, 0]


def kernel(u, i, j, user_embed, item_embed):
    tw_u = _tc_relayout(user_embed.T)
    tw_i = _tc_relayout(item_embed.T)
    d2 = _sc_pred_diff(u.astype(jnp.int32), i.astype(jnp.int32),
                       j.astype(jnp.int32), tw_u, tw_i)
    return _tc_loss(d2)


# speedup vs baseline: 4.2372x; 1.1660x over previous
import functools

import jax
import jax.numpy as jnp
from jax import lax
from jax.experimental import pallas as pl
from jax.experimental.pallas import tpu as pltpu
from jax.experimental.pallas import tpu_sc as plsc

NUM_CORES = 2
NUM_SUBCORES = 16
LANES = 16
NW = NUM_CORES * NUM_SUBCORES
BATCH = 16384
EDIM = 32
NROW = 1000000
WIDE = 128
PACK = WIDE // EDIM
WROWS = NROW * EDIM // WIDE
N4 = 253952
QB = 4096
NBLK = N4 // QB
INBLK = (NROW + QB - 1) // QB - 1
B_PER_W = BATCH // NW
CHUNK = 128
NCHUNK = B_PER_W // CHUNK
GROUPS = CHUNK // LANES


def _tc_relayout(t_t):
    def body(x0_ref, x1_ref, x2_ref, x3_ref, o_ref):
        for u, x_ref in enumerate((x0_ref, x1_ref, x2_ref, x3_ref)):
            o_ref[:, u * EDIM:(u + 1) * EDIM] = x_ref[...].T

    def in_spec(u):
        return pl.BlockSpec(
            (EDIM, QB),
            lambda g, u=u: (0, jnp.minimum(u * NBLK + g, INBLK)))

    return pl.pallas_call(
        body,
        grid=(NBLK,),
        in_specs=[in_spec(u) for u in range(4)],
        out_specs=pl.BlockSpec((QB, WIDE), lambda g: (g, 0)),
        out_shape=jax.ShapeDtypeStruct((N4, WIDE), jnp.float32),
    )(t_t, t_t, t_t, t_t)


def _sc_pred_diff(u, i, j, tw_u, tw_i):
    mesh = plsc.VectorSubcoreMesh(core_axis_name="c", subcore_axis_name="s")

    @functools.partial(
        pl.kernel,
        out_type=jax.ShapeDtypeStruct((BATCH // CHUNK, CHUNK), jnp.float32),
        mesh=mesh,
        compiler_params=pltpu.CompilerParams(
            needs_layout_passes=False, use_tc_tiling_on_sc=False),
        scratch_types=[
            pltpu.VMEM((NCHUNK, CHUNK), jnp.int32),
            pltpu.VMEM((NCHUNK, CHUNK), jnp.int32),
            pltpu.VMEM((NCHUNK, CHUNK), jnp.int32),
            pltpu.VMEM((NCHUNK, CHUNK), jnp.int32),
            pltpu.VMEM((NCHUNK, CHUNK), jnp.int32),
            pltpu.VMEM((NCHUNK, CHUNK), jnp.int32),
            pltpu.VMEM((CHUNK, WIDE), jnp.float32),
            pltpu.VMEM((CHUNK, WIDE), jnp.float32),
            pltpu.VMEM((CHUNK, WIDE), jnp.float32),
            pltpu.VMEM((NCHUNK, CHUNK), jnp.float32),
            pltpu.SemaphoreType.DMA,
        ],
    )
    def run(u_hbm, i_hbm, j_hbm, ut_hbm, it_hbm, out_hbm,
            u_idx, i_idx, j_idx, uq, iq, jq, ue_v, ie_v, je_v, pred_v, sem):
        wid = lax.axis_index("s") * NUM_CORES + lax.axis_index("c")
        base = wid * B_PER_W
        for c in range(NCHUNK):
            src = pl.ds(base + c * CHUNK, CHUNK)
            pltpu.sync_copy(u_hbm.at[src], u_idx.at[c])
            pltpu.sync_copy(i_hbm.at[src], i_idx.at[c])
            pltpu.sync_copy(j_hbm.at[src], j_idx.at[c])

        def qbody(v, carry):
            s = pl.ds(pl.multiple_of(v * LANES, LANES), LANES)
            for src_ref, dst_ref in ((u_idx, uq), (i_idx, iq), (j_idx, jq)):
                for c in range(NCHUNK):
                    r = src_ref[c, s]
                    slot = r // N4
                    dst_ref[c, s] = r - slot * N4
                    src_ref[c, s] = slot * EDIM
            return carry

        lax.fori_loop(0, CHUNK // LANES, qbody, 0)

        lane = lax.iota(jnp.int32, LANES)
        for c in range(NCHUNK):
            cp_u = pltpu.async_copy(ut_hbm.at[uq.at[c]], ue_v, sem)
            cp_i = pltpu.async_copy(it_hbm.at[iq.at[c]], ie_v, sem)
            cp_j = pltpu.async_copy(it_hbm.at[jq.at[c]], je_v, sem)
            cp_u.wait()
            cp_i.wait()
            cp_j.wait()

            def body(g, carry, c=c):
                s = pl.ds(pl.multiple_of(g * LANES, LANES), LANES)
                lrows = pl.multiple_of(g * LANES, LANES) + lane
                cb_u = u_idx[c, s]
                cb_i = i_idx[c, s]
                cb_j = j_idx[c, s]
                acc = jnp.zeros((LANES,), jnp.float32)
                for d in range(EDIM):
                    uev = plsc.load_gather(ue_v, [lrows, cb_u + d])
                    iev = plsc.load_gather(ie_v, [lrows, cb_i + d])
                    jev = plsc.load_gather(je_v, [lrows, cb_j + d])
                    acc = acc + uev * (iev - jev)
                pred_v[c, s] = acc
                return carry

            lax.fori_loop(0, GROUPS, body, 0)
        pltpu.sync_copy(pred_v, out_hbm.at[pl.ds(wid * NCHUNK, NCHUNK)])

    return run(u, i, j, tw_u, tw_i)


def _tc_loss(d2):
    def body(x_ref, o_ref):
        x = x_ref[...]
        ls = jnp.minimum(x, 0.0) - jnp.log(1.0 + jnp.exp(-jnp.abs(x)))
        o_ref[0, 0] = -jnp.sum(ls)

    out = pl.pallas_call(
        body,
        out_shape=jax.ShapeDtypeStruct((1, 1), jnp.float32),
        out_specs=pl.BlockSpec(memory_space=pltpu.SMEM),
    )(d2)
    return out[0, 0]


def kernel(u, i, j, user_embed, item_embed):
    tw_u = _tc_relayout(user_embed.T)
    tw_i = _tc_relayout(item_embed.T)
    d2 = _sc_pred_diff(u.astype(jnp.int32), i.astype(jnp.int32),
                       j.astype(jnp.int32), tw_u, tw_i)
    return _tc_loss(d2)
